# Initial kernel scaffold; baseline (speedup 1.0000x reference)
import jax
import jax.numpy as jnp
from jax.experimental import pallas as pl


def kernel(x, edge_index, batch, W1, a_src1, a_dst1, b1, W2, a_src2, a_dst2, b2, W3, a_src3, a_dst3, b3):
    raise NotImplementedError("write your pallas kernel here")


# speedup vs baseline: 1.3979x; 1.3979x over previous
import functools

import jax
import jax.numpy as jnp
from jax.experimental import pallas as pl
from jax.experimental.pallas import tpu as pltpu

N = 10000
E = 320000
P = E + N


def _mm_body(x_ref, w_ref, asrc_ref, adst_ref, h_ref, av_ref):
    x = x_ref[...]
    h = jnp.dot(x, w_ref[...], preferred_element_type=jnp.float32)
    h_ref[...] = h
    asrc = jnp.sum(h * asrc_ref[...], axis=1)
    adst = jnp.sum(h * adst_ref[...], axis=1)
    av_ref[...] = jnp.stack([asrc, adst], axis=1)


@functools.partial(jax.jit, static_argnames=("bm",))
def _mm(x, W, a_src, a_dst, bm=1000):
    n, k = x.shape
    f = W.shape[1]
    grid = (n // bm,)
    h, av = pl.pallas_call(
        _mm_body,
        grid=grid,
        in_specs=[
            pl.BlockSpec((bm, k), lambda i: (i, 0)),
            pl.BlockSpec((k, f), lambda i: (0, 0)),
            pl.BlockSpec((1, f), lambda i: (0, 0)),
            pl.BlockSpec((1, f), lambda i: (0, 0)),
        ],
        out_specs=[
            pl.BlockSpec((bm, f), lambda i: (i, 0)),
            pl.BlockSpec((bm, 2), lambda i: (i, 0)),
        ],
        out_shape=[
            jax.ShapeDtypeStruct((n, f), jnp.float32),
            jax.ShapeDtypeStruct((n, 2), jnp.float32),
        ],
    )(x, W, a_src.reshape(1, f), a_dst.reshape(1, f))
    return h, av[:, 0], av[:, 1]


def _edge_stage(h, asrc, adst, src, dst):
    e = asrc[src] + adst[dst]
    e = jnp.where(e >= 0, e, 0.2 * e)
    g = jnp.max(e)
    ex = jnp.exp(e - g)
    denom = jax.ops.segment_sum(ex, dst, num_segments=N)
    alpha = ex / (denom[dst] + 1e-16)
    return jax.ops.segment_sum(h[src] * alpha[:, None], dst, num_segments=N)


def kernel(x, edge_index, batch, W1, a_src1, a_dst1, b1,
           W2, a_src2, a_dst2, b2, W3, a_src3, a_dst3, b3):
    loop = jnp.arange(N, dtype=edge_index.dtype)
    src = jnp.concatenate([edge_index[0], loop])
    dst = jnp.concatenate([edge_index[1], loop])

    h, asrc, adst = _mm(x, W1, a_src1, a_dst1)
    a1 = _edge_stage(h, asrc, adst, src, dst) + b1
    a1 = jax.nn.relu(a1)

    h, asrc, adst = _mm(a1, W2, a_src2, a_dst2)
    a2 = _edge_stage(h, asrc, adst, src, dst) + b2
    a2 = jax.nn.relu(a2)

    h, asrc, adst = _mm(a2, W3, a_src3, a_dst3)
    return _edge_stage(h, asrc, adst, src, dst) + b3
